# baseline (speedup 1.0000x reference)
import jax
import jax.numpy as jnp
from jax.experimental import pallas as pl
from jax.experimental.pallas import tpu as pltpu

_A2, _A1, _A0 = -0.0058, 0.0324, 0.0141
_LN2 = 0.6931471805599453
_SIGN = 0x80000000

_R = 16
_B0, _B1 = 43, 86


def _gather_consts(jj):
    idx = {}
    for n in range(3):
        for k in range(3):
            p = (n + k) % 3
            sent = 1 if p == 0 else 0
            il = 3 * jj + (n - 128 * k)
            idx[(n, k)] = jnp.where((il >= 0) & (il < 128), il, sent)
    masks = [jnp.where(jj % 3 == p, 1.0, 0.0) for p in range(3)]
    return idx, masks


def _compact3_window(tw, idx, masks):
    acc = None
    for n in range(3):
        for k in range(3):
            p = (n + k) % 3
            xk = tw[:, 128 * k:128 * (k + 1)]
            g = jnp.take_along_axis(xk * masks[p], idx[(n, k)], axis=1)
            acc = g if acc is None else acc + g
    return acc


def _mc_body(u_ref, st_ref, rs_ref, mus_ref, std_ref, scale_ref, sum_ref, sq_ref):
    S = u_ref.shape[0]
    R = st_ref.shape[0]
    L = st_ref.shape[1]
    nch = L // 384
    jj = jnp.broadcast_to(jnp.arange(128, dtype=jnp.int32)[None, :], (R, 128))
    idx, masks = _gather_consts(jj)

    st = st_ref[...]
    scale_ref[...] = ((_A2 * st + _A1) * st + _A0) * (-_LN2)
    st_chunks = [
        _compact3_window(st[:, 384 * c:384 * (c + 1)], idx, masks)
        for c in range(nch)
    ]

    sum_ref[...] = jnp.zeros_like(sum_ref)
    sq_ref[...] = jnp.zeros_like(sq_ref)

    def step(s, carry):
        for c in range(nch):
            wsl = slice(384 * c, 384 * (c + 1))
            u = u_ref[s, :, wsl]
            v = u - 0.5
            a = jnp.abs(v)
            lg = jnp.log2(1.0 - (a + a))
            q = scale_ref[:, wsl] * lg
            sgn = pltpu.bitcast(v, jnp.uint32) & jnp.uint32(_SIGN)
            noise = pltpu.bitcast(pltpu.bitcast(q, jnp.uint32) | sgn,
                                  jnp.float32)
            y = _compact3_window(noise, idx, masks)
            sl = slice(128 * c, 128 * (c + 1))
            rs_ref[s, :, sl] = st_chunks[c] + y
            sum_ref[:, sl] += y
            sq_ref[:, sl] += y * y
        return carry

    jax.lax.fori_loop(0, S, step, 0, unroll=8)

    ssum = sum_ref[...]
    mean_y = ssum * (1.0 / S)
    var = (sq_ref[...] - ssum * mean_y) * (1.0 / (S - 1))
    std_ref[...] = jnp.sqrt(jnp.maximum(var, 0.0))
    mus_ref[...] = jnp.concatenate(st_chunks, axis=1) + mean_y


def kernel(current_state, u, no_sample):
    S, IN, OUT, N = u.shape
    L = OUT * N
    u3 = u.reshape(S, IN, L)
    st3 = current_state.reshape(IN, L)
    rs, mus, std = pl.pallas_call(
        _mc_body,
        grid=(IN // _R,),
        in_specs=[
            pl.BlockSpec((S, _R, L), lambda i: (0, i, 0)),
            pl.BlockSpec((_R, L), lambda i: (i, 0)),
        ],
        out_specs=[
            pl.BlockSpec((S, _R, OUT), lambda i: (0, i, 0)),
            pl.BlockSpec((_R, OUT), lambda i: (i, 0)),
            pl.BlockSpec((_R, OUT), lambda i: (i, 0)),
        ],
        out_shape=[
            jax.ShapeDtypeStruct((S, IN, OUT), jnp.float32),
            jax.ShapeDtypeStruct((IN, OUT), jnp.float32),
            jax.ShapeDtypeStruct((IN, OUT), jnp.float32),
        ],
        scratch_shapes=[
            pltpu.VMEM((_R, L), jnp.float32),
            pltpu.VMEM((_R, OUT), jnp.float32),
            pltpu.VMEM((_R, OUT), jnp.float32),
        ],
        compiler_params=pltpu.CompilerParams(
            dimension_semantics=("parallel",),
            vmem_limit_bytes=48 * 1024 * 1024,
        ),
        name="memristor_mc",
    )(u3, st3)
    return rs, mus, std


# speedup vs baseline: 1.2637x; 1.2637x over previous
import jax
import jax.numpy as jnp
from jax.experimental import pallas as pl
from jax.experimental.pallas import tpu as pltpu

_A2, _A1, _A0 = -0.0058, 0.0324, 0.0141
_LN2 = 0.6931471805599453
_SIGN = 0x80000000

_R = 16
_B0, _B1 = 43, 86


def _gather_consts(jj):
    idx = {}
    for n in range(3):
        for k in range(3):
            p = (n + k) % 3
            sent = 1 if p == 0 else 0
            il = 3 * jj + (n - 128 * k)
            idx[(n, k)] = jnp.where((il >= 0) & (il < 128), il, sent)
    masks = [jnp.where(jj % 3 == p, 1.0, 0.0) for p in range(3)]
    return idx, masks


def _compact3_window(tw, idx, masks):
    acc = None
    for n in range(3):
        for k in range(3):
            p = (n + k) % 3
            xk = tw[:, 128 * k:128 * (k + 1)]
            g = jnp.take_along_axis(xk * masks[p], idx[(n, k)], axis=1)
            acc = g if acc is None else acc + g
    return acc


def _mc_body(u_ref, st_ref, rs_ref, mus_ref, std_ref, scale_ref, nscale_ref,
             sum_ref, sq_ref):
    S = u_ref.shape[0]
    R = st_ref.shape[0]
    L = st_ref.shape[1]
    nch = L // 384
    jj = jnp.broadcast_to(jnp.arange(128, dtype=jnp.int32)[None, :], (R, 128))
    idx, masks = _gather_consts(jj)

    st = st_ref[...]
    sc_ln2 = ((_A2 * st + _A1) * st + _A0) * _LN2
    scale_ref[...] = sc_ln2
    nscale_ref[...] = -sc_ln2
    st_chunks = [
        _compact3_window(st[:, 384 * c:384 * (c + 1)], idx, masks)
        for c in range(nch)
    ]

    sum_ref[...] = jnp.zeros_like(sum_ref)
    sq_ref[...] = jnp.zeros_like(sq_ref)

    grp = min(4, nch)

    def step(s, carry):
        for g in range(nch // grp):
            cs = range(grp * g, grp * (g + 1))
            noises = []
            for c in cs:
                wsl = slice(384 * c, 384 * (c + 1))
                u = u_ref[s, :, wsl]
                v = u - 0.5
                a = jnp.abs(v)
                lg = jnp.log2(1.0 - (a + a))
                m = jnp.where(v < 0.0, scale_ref[:, wsl], nscale_ref[:, wsl])
                noises.append(m * lg)
            ys = [None] * grp
            for n in range(3):
                for k in range(3):
                    p = (n + k) % 3
                    for ci in range(grp):
                        xk = noises[ci][:, 128 * k:128 * (k + 1)]
                        gth = jnp.take_along_axis(
                            xk * masks[p], idx[(n, k)], axis=1)
                        ys[ci] = gth if ys[ci] is None else ys[ci] + gth
            for ci, c in enumerate(cs):
                sl = slice(128 * c, 128 * (c + 1))
                rs_ref[s, :, sl] = st_chunks[c] + ys[ci]
                sum_ref[:, sl] += ys[ci]
                sq_ref[:, sl] += ys[ci] * ys[ci]
        return carry

    jax.lax.fori_loop(0, S, step, 0, unroll=8)

    ssum = sum_ref[...]
    mean_y = ssum * (1.0 / S)
    var = (sq_ref[...] - ssum * mean_y) * (1.0 / (S - 1))
    std_ref[...] = jnp.sqrt(jnp.maximum(var, 0.0))
    mus_ref[...] = jnp.concatenate(st_chunks, axis=1) + mean_y


def kernel(current_state, u, no_sample):
    S, IN, OUT, N = u.shape
    L = OUT * N
    u3 = u.reshape(S, IN, L)
    st3 = current_state.reshape(IN, L)
    rs, mus, std = pl.pallas_call(
        _mc_body,
        grid=(IN // _R,),
        in_specs=[
            pl.BlockSpec((S, _R, L), lambda i: (0, i, 0)),
            pl.BlockSpec((_R, L), lambda i: (i, 0)),
        ],
        out_specs=[
            pl.BlockSpec((S, _R, OUT), lambda i: (0, i, 0)),
            pl.BlockSpec((_R, OUT), lambda i: (i, 0)),
            pl.BlockSpec((_R, OUT), lambda i: (i, 0)),
        ],
        out_shape=[
            jax.ShapeDtypeStruct((S, IN, OUT), jnp.float32),
            jax.ShapeDtypeStruct((IN, OUT), jnp.float32),
            jax.ShapeDtypeStruct((IN, OUT), jnp.float32),
        ],
        scratch_shapes=[
            pltpu.VMEM((_R, L), jnp.float32),
            pltpu.VMEM((_R, L), jnp.float32),
            pltpu.VMEM((_R, OUT), jnp.float32),
            pltpu.VMEM((_R, OUT), jnp.float32),
        ],
        compiler_params=pltpu.CompilerParams(
            dimension_semantics=("parallel",),
            vmem_limit_bytes=48 * 1024 * 1024,
        ),
        name="memristor_mc",
    )(u3, st3)
    return rs, mus, std


# speedup vs baseline: 7.1015x; 5.6194x over previous
import jax
import jax.numpy as jnp
from jax.experimental import pallas as pl
from jax.experimental.pallas import tpu as pltpu

_A2, _A1, _A0 = -0.0058, 0.0324, 0.0141
_LN2 = 0.6931471805599453

_R = 16


def _mc_body(u_ref, st_ref, rs_ref, mus_ref, std_ref, scale_ref, nscale_ref,
             sum_ref, sq_ref):
    S = u_ref.shape[0]
    N = u_ref.shape[1]

    st_sum = None
    for n in range(N):
        st = st_ref[n]
        sc_ln2 = ((_A2 * st + _A1) * st + _A0) * _LN2
        scale_ref[n] = sc_ln2
        nscale_ref[n] = -sc_ln2
        st_sum = st if st_sum is None else st_sum + st

    sum_ref[...] = jnp.zeros_like(sum_ref)
    sq_ref[...] = jnp.zeros_like(sq_ref)

    def step(s, carry):
        y = None
        for n in range(N):
            u = u_ref[s, n]
            v = u - 0.5
            a = jnp.abs(v)
            lg = jnp.log2(1.0 - (a + a))
            m = jnp.where(v < 0.0, scale_ref[n], nscale_ref[n])
            noise = m * lg
            y = noise if y is None else y + noise
        rs_ref[s] = st_sum + y
        sum_ref[...] += y
        sq_ref[...] += y * y
        return carry

    jax.lax.fori_loop(0, S, step, 0, unroll=8)

    ssum = sum_ref[...]
    mean_y = ssum * (1.0 / S)
    var = (sq_ref[...] - ssum * mean_y) * (1.0 / (S - 1))
    std_ref[...] = jnp.sqrt(jnp.maximum(var, 0.0))
    mus_ref[...] = st_sum + mean_y


def kernel(current_state, u, no_sample):
    S, IN, OUT, N = u.shape
    ut = jnp.transpose(u, (0, 3, 1, 2))
    stt = jnp.transpose(current_state, (2, 0, 1))
    rs, mus, std = pl.pallas_call(
        _mc_body,
        grid=(IN // _R,),
        in_specs=[
            pl.BlockSpec((S, N, _R, OUT), lambda i: (0, 0, i, 0)),
            pl.BlockSpec((N, _R, OUT), lambda i: (0, i, 0)),
        ],
        out_specs=[
            pl.BlockSpec((S, _R, OUT), lambda i: (0, i, 0)),
            pl.BlockSpec((_R, OUT), lambda i: (i, 0)),
            pl.BlockSpec((_R, OUT), lambda i: (i, 0)),
        ],
        out_shape=[
            jax.ShapeDtypeStruct((S, IN, OUT), jnp.float32),
            jax.ShapeDtypeStruct((IN, OUT), jnp.float32),
            jax.ShapeDtypeStruct((IN, OUT), jnp.float32),
        ],
        scratch_shapes=[
            pltpu.VMEM((N, _R, OUT), jnp.float32),
            pltpu.VMEM((N, _R, OUT), jnp.float32),
            pltpu.VMEM((_R, OUT), jnp.float32),
            pltpu.VMEM((_R, OUT), jnp.float32),
        ],
        compiler_params=pltpu.CompilerParams(
            dimension_semantics=("parallel",),
            vmem_limit_bytes=48 * 1024 * 1024,
        ),
        name="memristor_mc",
    )(ut, stt)
    return rs, mus, std


# speedup vs baseline: 7.1119x; 1.0015x over previous
import jax
import jax.numpy as jnp
from jax.experimental import pallas as pl
from jax.experimental.pallas import tpu as pltpu

_A2, _A1, _A0 = -0.0058, 0.0324, 0.0141
_LN2 = 0.6931471805599453

_R = 16


def _mc_body(u_ref, st_ref, rs_ref, mus_ref, std_ref, scale_ref, nscale_ref,
             sum_ref, sq_ref):
    S = u_ref.shape[0]
    N = u_ref.shape[1]

    st_sum = None
    for n in range(N):
        st = st_ref[n]
        sc_ln2 = ((_A2 * st + _A1) * st + _A0) * _LN2
        scale_ref[n] = sc_ln2
        nscale_ref[n] = -sc_ln2
        st_sum = st if st_sum is None else st_sum + st

    sum_ref[...] = jnp.zeros_like(sum_ref)
    sq_ref[...] = jnp.zeros_like(sq_ref)

    def one_sample(s):
        y = None
        for n in range(N):
            u = u_ref[s, n]
            v = u - 0.5
            a = jnp.abs(v)
            lg = jnp.log2(1.0 - (a + a))
            m = jnp.where(v < 0.0, scale_ref[n], nscale_ref[n])
            noise = m * lg
            y = noise if y is None else y + noise
        rs_ref[s] = st_sum + y
        return y

    def step(t, carry):
        s = t + t
        ya = one_sample(s)
        yb = one_sample(s + 1)
        sum_ref[...] += ya + yb
        sq_ref[...] += ya * ya + yb * yb
        return carry

    jax.lax.fori_loop(0, S // 2, step, 0, unroll=4)

    ssum = sum_ref[...]
    mean_y = ssum * (1.0 / S)
    var = (sq_ref[...] - ssum * mean_y) * (1.0 / (S - 1))
    std_ref[...] = jnp.sqrt(jnp.maximum(var, 0.0))
    mus_ref[...] = st_sum + mean_y


def kernel(current_state, u, no_sample):
    S, IN, OUT, N = u.shape
    ut = jnp.transpose(u, (0, 3, 1, 2))
    stt = jnp.transpose(current_state, (2, 0, 1))
    rs, mus, std = pl.pallas_call(
        _mc_body,
        grid=(IN // _R,),
        in_specs=[
            pl.BlockSpec((S, N, _R, OUT), lambda i: (0, 0, i, 0)),
            pl.BlockSpec((N, _R, OUT), lambda i: (0, i, 0)),
        ],
        out_specs=[
            pl.BlockSpec((S, _R, OUT), lambda i: (0, i, 0)),
            pl.BlockSpec((_R, OUT), lambda i: (i, 0)),
            pl.BlockSpec((_R, OUT), lambda i: (i, 0)),
        ],
        out_shape=[
            jax.ShapeDtypeStruct((S, IN, OUT), jnp.float32),
            jax.ShapeDtypeStruct((IN, OUT), jnp.float32),
            jax.ShapeDtypeStruct((IN, OUT), jnp.float32),
        ],
        scratch_shapes=[
            pltpu.VMEM((N, _R, OUT), jnp.float32),
            pltpu.VMEM((N, _R, OUT), jnp.float32),
            pltpu.VMEM((_R, OUT), jnp.float32),
            pltpu.VMEM((_R, OUT), jnp.float32),
        ],
        compiler_params=pltpu.CompilerParams(
            dimension_semantics=("parallel",),
            vmem_limit_bytes=48 * 1024 * 1024,
        ),
        name="memristor_mc",
    )(ut, stt)
    return rs, mus, std
